# baseline (speedup 1.0000x reference)
import jax
import jax.numpy as jnp
from jax import lax
from jax.experimental import pallas as pl
from jax.experimental.pallas import tpu as pltpu
from jax.experimental.pallas import tpu_sc as plsc

D = 256
NC, NS = 2, 16
NW = NC * NS
IC = 512


def _emb_body(src_tab, trg_tab, src_idx, trg_idx, src_out, trg_out,
              idx_s, isem0, isem1, dsem):
    B = src_idx.shape[0]
    bpw = B // NW
    nck = bpw // IC
    wid = lax.axis_index("s") * NC + lax.axis_index("c")
    base = wid * bpw
    isem = (isem0, isem1)

    for idx_hbm, tab_hbm, out_hbm in ((src_idx, src_tab, src_out),
                                      (trg_idx, trg_tab, trg_out)):

        def istart(slot, k):
            off = base + k * IC
            pltpu.async_copy(idx_hbm.at[pl.ds(off, IC)], idx_s.at[slot],
                             isem[slot])

        def iwait(slot, k):
            off = base + k * IC
            pltpu.make_async_copy(idx_hbm.at[pl.ds(off, IC)],
                                  idx_s.at[slot], isem[slot]).wait()

        istart(0, 0)
        istart(1, 1)

        def do_chunk(slot, k):
            iwait(slot, k)

            def row16(g, _):
                vec = idx_s[slot, pl.ds(g * 16, 16)]
                for j in range(16):
                    ridx = vec[j]
                    pltpu.async_copy(
                        tab_hbm.at[pl.ds(ridx, 1)],
                        out_hbm.at[pl.ds(base + k * IC + g * 16 + j, 1)],
                        dsem)
                return 0
            lax.fori_loop(0, IC // 16, row16, 0)

            @pl.when(k + 2 < nck)
            def _():
                istart(slot, k + 2)

        def pair(p, _):
            do_chunk(0, 2 * p)
            do_chunk(1, 2 * p + 1)
            return 0

        lax.fori_loop(0, nck // 2, pair, 0)

        pltpu.make_async_copy(tab_hbm.at[pl.ds(0, bpw)],
                              out_hbm.at[pl.ds(base, bpw)], dsem).wait()


def kernel(src_table, trg_table, src_indices, trg_indices):
    Bt, S = src_indices.shape
    B = Bt * S
    si = src_indices.reshape(B)
    ti = trg_indices.reshape(B)
    mesh = plsc.VectorSubcoreMesh(core_axis_name="c", subcore_axis_name="s",
                                  num_cores=NC, num_subcores=NS)
    k = pl.kernel(
        _emb_body,
        out_type=(jax.ShapeDtypeStruct((B, D), jnp.float32),
                  jax.ShapeDtypeStruct((B, D), jnp.float32)),
        mesh=mesh,
        scratch_types=[
            pltpu.VMEM((2, IC), jnp.int32),
            pltpu.SemaphoreType.DMA,
            pltpu.SemaphoreType.DMA,
            pltpu.SemaphoreType.DMA,
        ],
    )
    src_out, trg_out = k(src_table, trg_table, si, ti)
    return (src_out.reshape(Bt, S, D), trg_out.reshape(Bt, S, D))


# speedup vs baseline: 42.2848x; 42.2848x over previous
import jax
import jax.numpy as jnp
from jax import lax
from jax.experimental import pallas as pl
from jax.experimental.pallas import tpu as pltpu
from jax.experimental.pallas import tpu_sc as plsc

D = 256
NC, NS = 2, 16
NW = NC * NS
CHUNK = 80
NBUF = 5
NB2 = 2 * NBUF
K = 3


def _emb_body(src_tab, trg_tab, src_idx, trg_idx, src_out, trg_out,
              idx_v, rows_v, *sems):
    isem = sems[:NB2]
    gsem = sems[NB2:NB2 + NBUF]
    osem = sems[NB2 + NBUF:]
    B = src_idx.shape[0]
    bpw = B // NW
    nch = bpw // CHUNK
    nblocks = nch // NB2
    wid = lax.axis_index("s") * NC + lax.axis_index("c")
    base = wid * bpw

    for idx_hbm, tab_hbm, out_hbm in ((src_idx, src_tab, src_out),
                                      (trg_idx, trg_tab, trg_out)):

        def istart(sl, c):
            off = base + c * CHUNK
            pltpu.async_copy(idx_hbm.at[pl.ds(off, CHUNK)], idx_v.at[sl],
                             isem[sl])

        def iwait(sl, c):
            off = base + c * CHUNK
            pltpu.make_async_copy(idx_hbm.at[pl.ds(off, CHUNK)],
                                  idx_v.at[sl], isem[sl]).wait()

        def gstart(b, sl):
            pltpu.async_copy(tab_hbm.at[idx_v.at[sl]], rows_v.at[b], gsem[b])

        def gwait(b, sl):
            pltpu.make_async_copy(tab_hbm.at[idx_v.at[sl]], rows_v.at[b],
                                  gsem[b]).wait()

        def sstart(b, c):
            off = base + c * CHUNK
            pltpu.async_copy(rows_v.at[b], out_hbm.at[pl.ds(off, CHUNK)],
                             osem[b])

        def owait(b, c):
            off = base + c * CHUNK
            pltpu.make_async_copy(rows_v.at[b], out_hbm.at[pl.ds(off, CHUNK)],
                                  osem[b]).wait()

        for s in range(NBUF):
            istart(s, s)

        for s in range(NB2):
            if s >= NBUF:
                owait(s % NBUF, s - NBUF)
            iwait(s, s)
            gstart(s % NBUF, s)
            if s >= K:
                gwait((s - K) % NBUF, s - K)
                sstart((s - K) % NBUF, s - K)
            istart((s + NBUF) % NB2, s + NBUF)

        def block(r, _):
            g = r * NB2
            for s in range(NB2):
                c = g + s
                owait(s % NBUF, c - NBUF)
                iwait(s, c)
                gstart(s % NBUF, s)
                gwait((s - K) % NBUF, (s - K) % NB2)
                sstart((s - K) % NBUF, c - K)
                istart((s + NBUF) % NB2, c + NBUF)
            return 0

        lax.fori_loop(1, nblocks - 1, block, 0)

        g = (nblocks - 1) * NB2
        for s in range(NB2):
            c = g + s
            owait(s % NBUF, c - NBUF)
            iwait(s, c)
            gstart(s % NBUF, s)
            gwait((s - K) % NBUF, (s - K) % NB2)
            sstart((s - K) % NBUF, c - K)
            if c + NBUF < nch:
                istart((s + NBUF) % NB2, c + NBUF)

        for c in range(nch - K, nch):
            gwait(c % NBUF, c % NB2)
            sstart(c % NBUF, c)
        for c in range(nch - NBUF, nch):
            owait(c % NBUF, c)


def kernel(src_table, trg_table, src_indices, trg_indices):
    Bt, S = src_indices.shape
    B = Bt * S
    si = src_indices.reshape(B)
    ti = trg_indices.reshape(B)
    mesh = plsc.VectorSubcoreMesh(core_axis_name="c", subcore_axis_name="s",
                                  num_cores=NC, num_subcores=NS)
    k = pl.kernel(
        _emb_body,
        out_type=(jax.ShapeDtypeStruct((B, D), jnp.float32),
                  jax.ShapeDtypeStruct((B, D), jnp.float32)),
        mesh=mesh,
        scratch_types=(
            [pltpu.VMEM((NB2, CHUNK), jnp.int32),
             pltpu.VMEM((NBUF, CHUNK, D), jnp.float32)]
            + [pltpu.SemaphoreType.DMA] * (NB2 + 2 * NBUF)
        ),
    )
    src_out, trg_out = k(src_table, trg_table, si, ti)
    return (src_out.reshape(Bt, S, D), trg_out.reshape(Bt, S, D))
